# Initial kernel scaffold; baseline (speedup 1.0000x reference)
import jax
import jax.numpy as jnp
from jax.experimental import pallas as pl


def kernel(edge_index, edge_type, embedding, basis1, comp1, root1, bias1, basis2, comp2, root2, bias2):
    raise NotImplementedError("write your pallas kernel here")


# speedup vs baseline: 2.0619x; 2.0619x over previous
import functools

import jax
import jax.numpy as jnp
from jax import lax
from jax.experimental import pallas as pl
from jax.experimental.pallas import tpu as pltpu
from jax.experimental.pallas import tpu_sc as plsc

N = 10000
R = 50
D = 128
NB = 30
E = 320000

NC = 2
NS = 16
NW = NC * NS

E_PER_TILE = E // NW
CHUNK = 80
NCHUNK = E_PER_TILE // CHUNK

NSEG = N * R
NSEG_PAD = 512000
SEG_PER_TILE = NSEG_PAD // NS

N_PAD = 10240
ROWS_PER_TILE = N_PAD // NS
CP_CHUNK = 64
NCP = ROWS_PER_TILE // CP_CHUNK

SUP = 2000
NSUP = E_PER_TILE // SUP
NCH_SUP = SUP // CHUNK

NT_BLK = 1000
NT_GRID = N // NT_BLK

@functools.lru_cache(maxsize=None)
def _mesh():
    return plsc.VectorSubcoreMesh(core_axis_name="c", subcore_axis_name="s")


def _mix_body(comp_ref, basis_ref, w_ref):
    w_ref[0] = jnp.dot(comp_ref[0], basis_ref[0],
                       preferred_element_type=jnp.float32)


def _mix_weights(comp_s, basis_s):
    return pl.pallas_call(
        _mix_body,
        grid=(2,),
        in_specs=[
            pl.BlockSpec((1, R, NB), lambda i: (i, 0, 0)),
            pl.BlockSpec((1, NB, D * D), lambda i: (i, 0, 0)),
        ],
        out_specs=pl.BlockSpec((1, R, D * D), lambda i: (i, 0, 0)),
        out_shape=jax.ShapeDtypeStruct((2, R, D * D), jnp.float32),
    )(comp_s, basis_s)


def _h_body(x_ref, w_ref, h_ref):
    h_ref[0] = jnp.dot(x_ref[...], w_ref[0], preferred_element_type=jnp.float32)


def _h_all_relations(x, w):
    return pl.pallas_call(
        _h_body,
        grid=(NT_GRID, R),
        in_specs=[
            pl.BlockSpec((NT_BLK, D), lambda i, r: (i, 0)),
            pl.BlockSpec((1, D, D), lambda i, r: (r, 0, 0)),
        ],
        out_specs=pl.BlockSpec((1, NT_BLK, D), lambda i, r: (r, i, 0)),
        out_shape=jax.ShapeDtypeStruct((R, N, D), jnp.float32),
    )(x, w)


def _combine_body(relu, x_ref, p_ref, root_ref, bias_ref, o_ref):
    acc = p_ref[0] + p_ref[1] + bias_ref[0]
    acc = acc + jnp.dot(x_ref[...], root_ref[...],
                        preferred_element_type=jnp.float32)
    if relu:
        acc = jnp.maximum(acc, 0.0)
    o_ref[...] = acc


def _combine(x, p, root, bias2d, relu):
    return pl.pallas_call(
        functools.partial(_combine_body, relu),
        grid=(NT_GRID,),
        in_specs=[
            pl.BlockSpec((NT_BLK, D), lambda i: (i, 0)),
            pl.BlockSpec((2, NT_BLK, D), lambda i: (0, i, 0)),
            pl.BlockSpec((D, D), lambda i: (0, 0)),
            pl.BlockSpec((1, D), lambda i: (0, 0)),
        ],
        out_specs=pl.BlockSpec((NT_BLK, D), lambda i: (i, 0)),
        out_shape=jax.ShapeDtypeStruct((N, D), jnp.float32),
    )(x, p, root, bias2d)


def _zero_vmem(ref, nelem):
    def body(i, carry):
        ref[pl.ds(i * 16, 16)] = jnp.zeros((16,), jnp.float32)
        return carry
    lax.fori_loop(0, nelem // 16, body, 0)


def _deg_body(dst_hbm, typ_hbm, deg_hbm,
              dst_v, typ_v, key_v, ones_v, buf_v, acc_sh, sem):
    c = lax.axis_index("c")
    s = lax.axis_index("s")
    _zero_vmem(buf_v, SEG_PER_TILE)
    pltpu.sync_copy(buf_v, acc_sh.at[pl.ds(s * SEG_PER_TILE, SEG_PER_TILE)])

    def ob(i, carry):
        ones_v[pl.ds(i * 16, 16)] = jnp.ones((16,), jnp.float32)
        return carry
    lax.fori_loop(0, CHUNK // 16, ob, 0)
    plsc.subcore_barrier()

    base = (c * NS + s) * E_PER_TILE
    pltpu.sync_copy(dst_hbm.at[pl.ds(base, E_PER_TILE)], dst_v)
    pltpu.sync_copy(typ_hbm.at[pl.ds(base, E_PER_TILE)], typ_v)

    def chunk(ci, carry):
        off = ci * CHUNK

        def lane(g, carry2):
            key_v[pl.ds(g * 16, 16)] = (dst_v[pl.ds(off + g * 16, 16)] * R
                                        + typ_v[pl.ds(off + g * 16, 16)])
            return carry2
        lax.fori_loop(0, CHUNK // 16, lane, 0)
        pltpu.sync_copy(ones_v, acc_sh.at[key_v], add=True)
        return carry
    lax.fori_loop(0, NCHUNK, chunk, 0)
    plsc.subcore_barrier()

    pltpu.sync_copy(acc_sh.at[pl.ds(s * SEG_PER_TILE, SEG_PER_TILE)], buf_v)
    pltpu.sync_copy(buf_v, deg_hbm.at[c, pl.ds(s * SEG_PER_TILE, SEG_PER_TILE)])


def _deg_counts(dst, typ):
    return pl.kernel(
        _deg_body,
        mesh=_mesh(),
        out_type=jax.ShapeDtypeStruct((NC, NSEG_PAD), jnp.float32),
        scratch_types=[
            pltpu.VMEM((E_PER_TILE,), jnp.int32),
            pltpu.VMEM((E_PER_TILE,), jnp.int32),
            pltpu.VMEM((CHUNK,), jnp.int32),
            pltpu.VMEM((CHUNK,), jnp.float32),
            pltpu.VMEM((SEG_PER_TILE,), jnp.float32),
            pltpu.VMEM_SHARED((NSEG_PAD,), jnp.float32),
            pltpu.SemaphoreType.DMA,
        ],
    )(dst, typ)


def _edges_body(compute_w, src_hbm, dst_hbm, typ_hbm, wd0_hbm, wd1_hbm, h_hbm,
                p_hbm, w_hbm,
                src_v, dst_v, typ_v, w_v, keyc_v, idxc_v, dstc_v,
                d0_v, d1_v, rows_v, cbuf_v, acc_sh, sem):
    c = lax.axis_index("c")
    s = lax.axis_index("s")
    base = (c * NS + s) * E_PER_TILE

    def zrow(i, carry):
        for j in range(D // 16):
            cbuf_v[i, pl.ds(j * 16, 16)] = jnp.zeros((16,), jnp.float32)
        return carry
    lax.fori_loop(0, CP_CHUNK, zrow, 0)
    for j in range(NCP):
        pltpu.sync_copy(
            cbuf_v,
            acc_sh.at[pl.ds(s * ROWS_PER_TILE + j * CP_CHUNK, CP_CHUNK)])
    plsc.subcore_barrier()

    def sup(si, carry):
        sbase = base + si * SUP
        pltpu.sync_copy(src_hbm.at[pl.ds(sbase, SUP)], src_v)
        pltpu.sync_copy(dst_hbm.at[pl.ds(sbase, SUP)], dst_v)
        pltpu.sync_copy(typ_hbm.at[pl.ds(sbase, SUP)], typ_v)

        if compute_w:
            def wchunk(ci, carry2):
                off = ci * CHUNK

                def lane(g, c3):
                    keyc_v[pl.ds(g * 16, 16)] = (
                        dst_v[pl.ds(off + g * 16, 16)] * R
                        + typ_v[pl.ds(off + g * 16, 16)])
                    return c3
                lax.fori_loop(0, CHUNK // 16, lane, 0)
                pltpu.async_copy(wd0_hbm.at[keyc_v], d0_v, sem).wait()
                pltpu.async_copy(wd1_hbm.at[keyc_v], d1_v, sem).wait()

                def lane2(g, c3):
                    dsum = d0_v[pl.ds(g * 16, 16)] + d1_v[pl.ds(g * 16, 16)]
                    w_v[pl.ds(off + g * 16, 16)] = 1.0 / jnp.maximum(dsum, 1.0)
                    return c3
                lax.fori_loop(0, CHUNK // 16, lane2, 0)
                return carry2
            lax.fori_loop(0, NCH_SUP, wchunk, 0)
            pltpu.sync_copy(w_v, w_hbm.at[pl.ds(sbase, SUP)])
        else:
            pltpu.sync_copy(wd0_hbm.at[pl.ds(sbase, SUP)], w_v)

        def chunk(ci, carry2):
            off = ci * CHUNK

            def lane(g, c3):
                idxc_v[pl.ds(g * 16, 16)] = (
                    typ_v[pl.ds(off + g * 16, 16)] * N
                    + src_v[pl.ds(off + g * 16, 16)])
                dstc_v[pl.ds(g * 16, 16)] = dst_v[pl.ds(off + g * 16, 16)]
                return c3
            lax.fori_loop(0, CHUNK // 16, lane, 0)

            pltpu.async_copy(h_hbm.at[idxc_v], rows_v, sem).wait()

            def escale(g, c3):
                wvec = w_v[pl.ds(off + g * 16, 16)]
                for i in range(16):
                    wv = wvec[i]
                    e = g * 16 + i
                    for j in range(D // 16):
                        rows_v[e, pl.ds(j * 16, 16)] = (
                            rows_v[e, pl.ds(j * 16, 16)] * wv)
                return c3
            lax.fori_loop(0, CHUNK // 16, escale, 0)

            pltpu.sync_copy(rows_v, acc_sh.at[dstc_v], add=True)
            return carry2
        lax.fori_loop(0, NCH_SUP, chunk, 0)
        return carry
    lax.fori_loop(0, NSUP, sup, 0)
    plsc.subcore_barrier()

    for j in range(NCP):
        row0 = s * ROWS_PER_TILE + j * CP_CHUNK
        pltpu.sync_copy(acc_sh.at[pl.ds(row0, CP_CHUNK)], cbuf_v)
        pltpu.sync_copy(cbuf_v, p_hbm.at[c, pl.ds(row0, CP_CHUNK)])


def _edge_aggregate(src, dst, typ, wd0, wd1, hflat, compute_w):
    return pl.kernel(
        functools.partial(_edges_body, compute_w),
        mesh=_mesh(),
        out_type=(
            jax.ShapeDtypeStruct((NC, N_PAD, D), jnp.float32),
            jax.ShapeDtypeStruct((E,), jnp.float32),
        ),
        scratch_types=[
            pltpu.VMEM((SUP,), jnp.int32),
            pltpu.VMEM((SUP,), jnp.int32),
            pltpu.VMEM((SUP,), jnp.int32),
            pltpu.VMEM((SUP,), jnp.float32),
            pltpu.VMEM((CHUNK,), jnp.int32),
            pltpu.VMEM((CHUNK,), jnp.int32),
            pltpu.VMEM((CHUNK,), jnp.int32),
            pltpu.VMEM((CHUNK,), jnp.float32),
            pltpu.VMEM((CHUNK,), jnp.float32),
            pltpu.VMEM((CHUNK, D), jnp.float32),
            pltpu.VMEM((CP_CHUNK, D), jnp.float32),
            pltpu.VMEM_SHARED((N_PAD, D), jnp.float32),
            pltpu.SemaphoreType.DMA,
        ],
    )(src, dst, typ, wd0, wd1, hflat)


def kernel(edge_index, edge_type, embedding,
           basis1, comp1, root1, bias1,
           basis2, comp2, root2, bias2):
    src = edge_index[0].astype(jnp.int32)
    dst = edge_index[1].astype(jnp.int32)
    typ = edge_type.astype(jnp.int32)

    comp_s = jnp.stack([comp1, comp2])
    basis_s = jnp.stack([basis1.reshape(NB, D * D),
                         basis2.reshape(NB, D * D)])
    w_mix = _mix_weights(comp_s, basis_s)
    w1 = w_mix[0].reshape(R, D, D)
    w2 = w_mix[1].reshape(R, D, D)

    deg = _deg_counts(dst, typ)

    h1 = _h_all_relations(embedding, w1).reshape(R * N, D)
    p1, ew = _edge_aggregate(src, dst, typ, deg[0], deg[1], h1, compute_w=True)
    p1 = p1[:, :N]
    x2 = _combine(embedding, p1, root1, bias1.reshape(1, D), relu=True)

    h2 = _h_all_relations(x2, w2).reshape(R * N, D)
    p2, _ = _edge_aggregate(src, dst, typ, ew, ew, h2, compute_w=False)
    p2 = p2[:, :N]
    out = _combine(x2, p2, root2, bias2.reshape(1, D), relu=False)
    return out


# speedup vs baseline: 2.7805x; 1.3485x over previous
import functools

import jax
import jax.numpy as jnp
from jax import lax
from jax.experimental import pallas as pl
from jax.experimental.pallas import tpu as pltpu
from jax.experimental.pallas import tpu_sc as plsc

N = 10000
R = 50
D = 128
NB = 30
E = 320000

NC = 2
NS = 16
NW = NC * NS

E_PER_TILE = E // NW
CHUNK = 80
NCHUNK = E_PER_TILE // CHUNK

NSEG = N * R
NSEG_PAD = 512000
SEG_PER_TILE = NSEG_PAD // NS

N_PAD = 10240
ROWS_PER_TILE = N_PAD // NS
CP_CHUNK = 64
NCP = ROWS_PER_TILE // CP_CHUNK

SUP = 2000
NSUP = E_PER_TILE // SUP
NCH_SUP = SUP // CHUNK

NT_BLK = 1000
NT_GRID = N // NT_BLK

@functools.lru_cache(maxsize=None)
def _mesh():
    return plsc.VectorSubcoreMesh(core_axis_name="c", subcore_axis_name="s")


def _mix_body(comp_ref, basis_ref, w_ref):
    w_ref[0] = jnp.dot(comp_ref[0], basis_ref[0],
                       preferred_element_type=jnp.float32)


def _mix_weights(comp_s, basis_s):
    return pl.pallas_call(
        _mix_body,
        grid=(2,),
        in_specs=[
            pl.BlockSpec((1, R, NB), lambda i: (i, 0, 0)),
            pl.BlockSpec((1, NB, D * D), lambda i: (i, 0, 0)),
        ],
        out_specs=pl.BlockSpec((1, R, D * D), lambda i: (i, 0, 0)),
        out_shape=jax.ShapeDtypeStruct((2, R, D * D), jnp.float32),
    )(comp_s, basis_s)


def _h_body(x_ref, w_ref, h_ref):
    h_ref[0] = jnp.dot(x_ref[...], w_ref[0], preferred_element_type=jnp.float32)


def _h_all_relations(x, w):
    return pl.pallas_call(
        _h_body,
        grid=(NT_GRID, R),
        in_specs=[
            pl.BlockSpec((NT_BLK, D), lambda i, r: (i, 0)),
            pl.BlockSpec((1, D, D), lambda i, r: (r, 0, 0)),
        ],
        out_specs=pl.BlockSpec((1, NT_BLK, D), lambda i, r: (r, i, 0)),
        out_shape=jax.ShapeDtypeStruct((R, N, D), jnp.float32),
    )(x, w)


def _combine_body(relu, x_ref, p_ref, root_ref, bias_ref, o_ref):
    acc = p_ref[0] + p_ref[1] + bias_ref[0]
    acc = acc + jnp.dot(x_ref[...], root_ref[...],
                        preferred_element_type=jnp.float32)
    if relu:
        acc = jnp.maximum(acc, 0.0)
    o_ref[...] = acc


def _combine(x, p, root, bias2d, relu):
    return pl.pallas_call(
        functools.partial(_combine_body, relu),
        grid=(NT_GRID,),
        in_specs=[
            pl.BlockSpec((NT_BLK, D), lambda i: (i, 0)),
            pl.BlockSpec((2, NT_BLK, D), lambda i: (0, i, 0)),
            pl.BlockSpec((D, D), lambda i: (0, 0)),
            pl.BlockSpec((1, D), lambda i: (0, 0)),
        ],
        out_specs=pl.BlockSpec((NT_BLK, D), lambda i: (i, 0)),
        out_shape=jax.ShapeDtypeStruct((N, D), jnp.float32),
    )(x, p, root, bias2d)


def _zero_vmem(ref, nelem):
    def body(i, carry):
        ref[pl.ds(i * 16, 16)] = jnp.zeros((16,), jnp.float32)
        return carry
    lax.fori_loop(0, nelem // 16, body, 0)


def _deg_body(dst_hbm, typ_hbm, deg_hbm,
              dst_v, typ_v, key_v, ones_v, buf_v, acc_sh, sem):
    c = lax.axis_index("c")
    s = lax.axis_index("s")
    _zero_vmem(buf_v, SEG_PER_TILE)
    pltpu.sync_copy(buf_v, acc_sh.at[pl.ds(s * SEG_PER_TILE, SEG_PER_TILE)])

    def ob(i, carry):
        ones_v[pl.ds(i * 16, 16)] = jnp.ones((16,), jnp.float32)
        return carry
    lax.fori_loop(0, CHUNK // 16, ob, 0)
    plsc.subcore_barrier()

    base = (c * NS + s) * E_PER_TILE
    pltpu.sync_copy(dst_hbm.at[pl.ds(base, E_PER_TILE)], dst_v)
    pltpu.sync_copy(typ_hbm.at[pl.ds(base, E_PER_TILE)], typ_v)

    def chunk(ci, carry):
        off = ci * CHUNK

        def lane(g, carry2):
            key_v[pl.ds(g * 16, 16)] = (dst_v[pl.ds(off + g * 16, 16)] * R
                                        + typ_v[pl.ds(off + g * 16, 16)])
            return carry2
        lax.fori_loop(0, CHUNK // 16, lane, 0)
        pltpu.sync_copy(ones_v, acc_sh.at[key_v], add=True)
        return carry
    lax.fori_loop(0, NCHUNK, chunk, 0)
    plsc.subcore_barrier()

    pltpu.sync_copy(acc_sh.at[pl.ds(s * SEG_PER_TILE, SEG_PER_TILE)], buf_v)
    pltpu.sync_copy(buf_v, deg_hbm.at[c, pl.ds(s * SEG_PER_TILE, SEG_PER_TILE)])


def _deg_counts(dst, typ):
    return pl.kernel(
        _deg_body,
        mesh=_mesh(),
        out_type=jax.ShapeDtypeStruct((NC, NSEG_PAD), jnp.float32),
        scratch_types=[
            pltpu.VMEM((E_PER_TILE,), jnp.int32),
            pltpu.VMEM((E_PER_TILE,), jnp.int32),
            pltpu.VMEM((CHUNK,), jnp.int32),
            pltpu.VMEM((CHUNK,), jnp.float32),
            pltpu.VMEM((SEG_PER_TILE,), jnp.float32),
            pltpu.VMEM_SHARED((NSEG_PAD,), jnp.float32),
            pltpu.SemaphoreType.DMA,
        ],
    )(dst, typ)


def _edges_body(compute_w, src_hbm, dst_hbm, typ_hbm, wd0_hbm, wd1_hbm, h_hbm,
                p_hbm, w_hbm,
                src_v, dst_v, typ_v, w_v,
                idxc_a, idxc_b, dstc_a, dstc_b, keyc_a, keyc_b,
                d0_a, d1_a, d0_b, d1_b, rows_a, rows_b, cbuf_v, acc_sh,
                g_a, g_b, sd0_a, sd1_a, sd0_b, sd1_b):
    c = lax.axis_index("c")
    s = lax.axis_index("s")
    base = (c * NS + s) * E_PER_TILE

    def zrow(i, carry):
        for j in range(D // 16):
            cbuf_v[i, pl.ds(j * 16, 16)] = jnp.zeros((16,), jnp.float32)
        return carry
    lax.fori_loop(0, CP_CHUNK, zrow, 0)
    for j in range(NCP):
        pltpu.sync_copy(
            cbuf_v,
            acc_sh.at[pl.ds(s * ROWS_PER_TILE + j * CP_CHUNK, CP_CHUNK)])
    plsc.subcore_barrier()

    def fire(off, idxc_v, dstc_v, keyc_v, rows_v, g, s0, s1, d0_v, d1_v):
        def lane(g_, c3):
            sl = pl.ds(off + g_ * 16, 16)
            ol = pl.ds(g_ * 16, 16)
            t16 = typ_v[sl]
            idxc_v[ol] = t16 * N + src_v[sl]
            dstc_v[ol] = dst_v[sl]
            if compute_w:
                keyc_v[ol] = dst_v[sl] * R + t16
            return c3
        lax.fori_loop(0, CHUNK // 16, lane, 0)
        pltpu.async_copy(h_hbm.at[idxc_v], rows_v, g)
        if compute_w:
            pltpu.async_copy(wd0_hbm.at[keyc_v], d0_v, s0)
            pltpu.async_copy(wd1_hbm.at[keyc_v], d1_v, s1)

    def process(off, idxc_v, dstc_v, keyc_v, rows_v, g, s0, s1, d0_v, d1_v):
        if compute_w:
            pltpu.make_async_copy(wd0_hbm.at[keyc_v], d0_v, s0).wait()
            pltpu.make_async_copy(wd1_hbm.at[keyc_v], d1_v, s1).wait()

            def lw(g_, c3):
                dsum = d0_v[pl.ds(g_ * 16, 16)] + d1_v[pl.ds(g_ * 16, 16)]
                w_v[pl.ds(off + g_ * 16, 16)] = 1.0 / jnp.maximum(dsum, 1.0)
                return c3
            lax.fori_loop(0, CHUNK // 16, lw, 0)
        pltpu.make_async_copy(h_hbm.at[idxc_v], rows_v, g).wait()

        def escale(g_, c3):
            wvec = w_v[pl.ds(off + g_ * 16, 16)]
            for i in range(16):
                wv = wvec[i]
                e = g_ * 16 + i
                for j in range(D // 16):
                    rows_v[e, pl.ds(j * 16, 16)] = (
                        rows_v[e, pl.ds(j * 16, 16)] * wv)
            return c3
        lax.fori_loop(0, CHUNK // 16, escale, 0)
        pltpu.sync_copy(rows_v, acc_sh.at[dstc_v], add=True)

    A = (idxc_a, dstc_a, keyc_a, rows_a, g_a, sd0_a, sd1_a, d0_a, d1_a)
    B = (idxc_b, dstc_b, keyc_b, rows_b, g_b, sd0_b, sd1_b, d0_b, d1_b)

    def fire_t(off, t):
        fire(off, t[0], t[1], t[2], t[3], t[4], t[5], t[6], t[7], t[8])

    def process_t(off, t):
        process(off, t[0], t[1], t[2], t[3], t[4], t[5], t[6], t[7], t[8])

    def sup(si, carry):
        sbase = base + si * SUP
        pltpu.sync_copy(src_hbm.at[pl.ds(sbase, SUP)], src_v)
        pltpu.sync_copy(dst_hbm.at[pl.ds(sbase, SUP)], dst_v)
        pltpu.sync_copy(typ_hbm.at[pl.ds(sbase, SUP)], typ_v)
        if not compute_w:
            pltpu.sync_copy(wd0_hbm.at[pl.ds(sbase, SUP)], w_v)

        fire_t(0, A)

        def pair(k, c2):
            fire_t((2 * k + 1) * CHUNK, B)
            process_t(2 * k * CHUNK, A)
            fire_t((2 * k + 2) * CHUNK, A)
            process_t((2 * k + 1) * CHUNK, B)
            return c2
        lax.fori_loop(0, (NCH_SUP - 1) // 2, pair, 0)
        process_t((NCH_SUP - 1) * CHUNK, A)

        if compute_w:
            pltpu.sync_copy(w_v, w_hbm.at[pl.ds(sbase, SUP)])
        return carry
    lax.fori_loop(0, NSUP, sup, 0)
    plsc.subcore_barrier()

    for j in range(NCP):
        row0 = s * ROWS_PER_TILE + j * CP_CHUNK
        pltpu.sync_copy(acc_sh.at[pl.ds(row0, CP_CHUNK)], cbuf_v)
        pltpu.sync_copy(cbuf_v, p_hbm.at[c, pl.ds(row0, CP_CHUNK)])


def _edge_aggregate(src, dst, typ, wd0, wd1, hflat, compute_w):
    return pl.kernel(
        functools.partial(_edges_body, compute_w),
        mesh=_mesh(),
        out_type=(
            jax.ShapeDtypeStruct((NC, N_PAD, D), jnp.float32),
            jax.ShapeDtypeStruct((E,), jnp.float32),
        ),
        scratch_types=[
            pltpu.VMEM((SUP,), jnp.int32),
            pltpu.VMEM((SUP,), jnp.int32),
            pltpu.VMEM((SUP,), jnp.int32),
            pltpu.VMEM((SUP,), jnp.float32),
            pltpu.VMEM((CHUNK,), jnp.int32),
            pltpu.VMEM((CHUNK,), jnp.int32),
            pltpu.VMEM((CHUNK,), jnp.int32),
            pltpu.VMEM((CHUNK,), jnp.int32),
            pltpu.VMEM((CHUNK,), jnp.int32),
            pltpu.VMEM((CHUNK,), jnp.int32),
            pltpu.VMEM((CHUNK,), jnp.float32),
            pltpu.VMEM((CHUNK,), jnp.float32),
            pltpu.VMEM((CHUNK,), jnp.float32),
            pltpu.VMEM((CHUNK,), jnp.float32),
            pltpu.VMEM((CHUNK, D), jnp.float32),
            pltpu.VMEM((CHUNK, D), jnp.float32),
            pltpu.VMEM((CP_CHUNK, D), jnp.float32),
            pltpu.VMEM_SHARED((N_PAD, D), jnp.float32),
            pltpu.SemaphoreType.DMA,
            pltpu.SemaphoreType.DMA,
            pltpu.SemaphoreType.DMA,
            pltpu.SemaphoreType.DMA,
            pltpu.SemaphoreType.DMA,
            pltpu.SemaphoreType.DMA,
        ],
    )(src, dst, typ, wd0, wd1, hflat)


def kernel(edge_index, edge_type, embedding,
           basis1, comp1, root1, bias1,
           basis2, comp2, root2, bias2):
    src = edge_index[0].astype(jnp.int32)
    dst = edge_index[1].astype(jnp.int32)
    typ = edge_type.astype(jnp.int32)

    comp_s = jnp.stack([comp1, comp2])
    basis_s = jnp.stack([basis1.reshape(NB, D * D),
                         basis2.reshape(NB, D * D)])
    w_mix = _mix_weights(comp_s, basis_s)
    w1 = w_mix[0].reshape(R, D, D)
    w2 = w_mix[1].reshape(R, D, D)

    deg = _deg_counts(dst, typ)

    h1 = _h_all_relations(embedding, w1).reshape(R * N, D)
    p1, ew = _edge_aggregate(src, dst, typ, deg[0], deg[1], h1, compute_w=True)
    p1 = p1[:, :N]
    x2 = _combine(embedding, p1, root1, bias1.reshape(1, D), relu=True)

    h2 = _h_all_relations(x2, w2).reshape(R * N, D)
    p2, _ = _edge_aggregate(src, dst, typ, ew, ew, h2, compute_w=False)
    p2 = p2[:, :N]
    out = _combine(x2, p2, root2, bias2.reshape(1, D), relu=False)
    return out


# speedup vs baseline: 2.8539x; 1.0264x over previous
import functools

import jax
import jax.numpy as jnp
from jax import lax
from jax.experimental import pallas as pl
from jax.experimental.pallas import tpu as pltpu
from jax.experimental.pallas import tpu_sc as plsc

N = 10000
R = 50
D = 128
NB = 30
E = 320000

NC = 2
NS = 16
NW = NC * NS

E_PER_TILE = E // NW
CHUNK = 80
NCHUNK = E_PER_TILE // CHUNK

NSEG = N * R
NSEG_PAD = 512000
SEG_PER_TILE = NSEG_PAD // NS

N_PAD = 10240
ROWS_PER_TILE = N_PAD // NS
CP_CHUNK = 64
NCP = ROWS_PER_TILE // CP_CHUNK

SUP = 2000
NSUP = E_PER_TILE // SUP
NCH_SUP = SUP // CHUNK

NT_BLK = 1000
NT_GRID = N // NT_BLK

@functools.lru_cache(maxsize=None)
def _mesh():
    return plsc.VectorSubcoreMesh(core_axis_name="c", subcore_axis_name="s")


def _mix_body(comp_ref, basis_ref, w_ref):
    w_ref[0] = jnp.dot(comp_ref[0], basis_ref[0],
                       preferred_element_type=jnp.float32)


def _mix_weights(comp_s, basis_s):
    return pl.pallas_call(
        _mix_body,
        grid=(2,),
        in_specs=[
            pl.BlockSpec((1, R, NB), lambda i: (i, 0, 0)),
            pl.BlockSpec((1, NB, D * D), lambda i: (i, 0, 0)),
        ],
        out_specs=pl.BlockSpec((1, R, D * D), lambda i: (i, 0, 0)),
        out_shape=jax.ShapeDtypeStruct((2, R, D * D), jnp.float32),
    )(comp_s, basis_s)


def _h_body(x_ref, w_ref, h_ref):
    h_ref[0] = jnp.dot(x_ref[...], w_ref[0], preferred_element_type=jnp.float32)


def _h_all_relations(x, w):
    return pl.pallas_call(
        _h_body,
        grid=(NT_GRID, R),
        in_specs=[
            pl.BlockSpec((NT_BLK, D), lambda i, r: (i, 0)),
            pl.BlockSpec((1, D, D), lambda i, r: (r, 0, 0)),
        ],
        out_specs=pl.BlockSpec((1, NT_BLK, D), lambda i, r: (r, i, 0)),
        out_shape=jax.ShapeDtypeStruct((R, N, D), jnp.float32),
    )(x, w)


def _combine_body(relu, x_ref, p_ref, root_ref, bias_ref, o_ref):
    acc = p_ref[0] + p_ref[1] + bias_ref[0]
    acc = acc + jnp.dot(x_ref[...], root_ref[...],
                        preferred_element_type=jnp.float32)
    if relu:
        acc = jnp.maximum(acc, 0.0)
    o_ref[...] = acc


def _combine(x, p, root, bias2d, relu):
    return pl.pallas_call(
        functools.partial(_combine_body, relu),
        grid=(NT_GRID,),
        in_specs=[
            pl.BlockSpec((NT_BLK, D), lambda i: (i, 0)),
            pl.BlockSpec((2, NT_BLK, D), lambda i: (0, i, 0)),
            pl.BlockSpec((D, D), lambda i: (0, 0)),
            pl.BlockSpec((1, D), lambda i: (0, 0)),
        ],
        out_specs=pl.BlockSpec((NT_BLK, D), lambda i: (i, 0)),
        out_shape=jax.ShapeDtypeStruct((N, D), jnp.float32),
    )(x, p, root, bias2d)


def _zero_vmem(ref, nelem):
    def body(i, carry):
        ref[pl.ds(i * 16, 16)] = jnp.zeros((16,), jnp.float32)
        return carry
    lax.fori_loop(0, nelem // 16, body, 0)


def _deg_body(dst_hbm, typ_hbm, deg_hbm,
              dst_v, typ_v, key_v, ones_v, buf_v, acc_sh, sem):
    c = lax.axis_index("c")
    s = lax.axis_index("s")
    _zero_vmem(buf_v, SEG_PER_TILE)
    pltpu.sync_copy(buf_v, acc_sh.at[pl.ds(s * SEG_PER_TILE, SEG_PER_TILE)])

    def ob(i, carry):
        ones_v[pl.ds(i * 16, 16)] = jnp.ones((16,), jnp.float32)
        return carry
    lax.fori_loop(0, CHUNK // 16, ob, 0)
    plsc.subcore_barrier()

    base = (c * NS + s) * E_PER_TILE
    pltpu.sync_copy(dst_hbm.at[pl.ds(base, E_PER_TILE)], dst_v)
    pltpu.sync_copy(typ_hbm.at[pl.ds(base, E_PER_TILE)], typ_v)

    def chunk(ci, carry):
        off = ci * CHUNK

        def lane(g, carry2):
            key_v[pl.ds(g * 16, 16)] = (dst_v[pl.ds(off + g * 16, 16)] * R
                                        + typ_v[pl.ds(off + g * 16, 16)])
            return carry2
        lax.fori_loop(0, CHUNK // 16, lane, 0)
        pltpu.sync_copy(ones_v, acc_sh.at[key_v], add=True)
        return carry
    lax.fori_loop(0, NCHUNK, chunk, 0)
    plsc.subcore_barrier()

    pltpu.sync_copy(acc_sh.at[pl.ds(s * SEG_PER_TILE, SEG_PER_TILE)], buf_v)
    pltpu.sync_copy(buf_v, deg_hbm.at[c, pl.ds(s * SEG_PER_TILE, SEG_PER_TILE)])


def _deg_counts(dst, typ):
    return pl.kernel(
        _deg_body,
        mesh=_mesh(),
        out_type=jax.ShapeDtypeStruct((NC, NSEG_PAD), jnp.float32),
        scratch_types=[
            pltpu.VMEM((E_PER_TILE,), jnp.int32),
            pltpu.VMEM((E_PER_TILE,), jnp.int32),
            pltpu.VMEM((CHUNK,), jnp.int32),
            pltpu.VMEM((CHUNK,), jnp.float32),
            pltpu.VMEM((SEG_PER_TILE,), jnp.float32),
            pltpu.VMEM_SHARED((NSEG_PAD,), jnp.float32),
            pltpu.SemaphoreType.DMA,
        ],
    )(dst, typ)


def _edges_body(compute_w, src_hbm, dst_hbm, typ_hbm, wd0_hbm, wd1_hbm, h_hbm,
                p_hbm, w_hbm,
                src_v, dst_v, typ_v, w_v,
                idxc_a, idxc_b, dstc_a, dstc_b, keyc_a, keyc_b,
                d0_a, d1_a, d0_b, d1_b, rows_a, rows_b, cbuf_v, acc_sh,
                g_a, g_b, sd0_a, sd1_a, sd0_b, sd1_b):
    c = lax.axis_index("c")
    s = lax.axis_index("s")
    base = (c * NS + s) * E_PER_TILE

    def zrow(i, carry):
        for j in range(D // 16):
            cbuf_v[i, pl.ds(j * 16, 16)] = jnp.zeros((16,), jnp.float32)
        return carry
    lax.fori_loop(0, CP_CHUNK, zrow, 0)
    for j in range(NCP):
        pltpu.sync_copy(
            cbuf_v,
            acc_sh.at[pl.ds(s * ROWS_PER_TILE + j * CP_CHUNK, CP_CHUNK)])
    plsc.subcore_barrier()

    def fire(off, idxc_v, dstc_v, keyc_v, rows_v, g, s0, s1, d0_v, d1_v):
        def lane(g_, c3):
            sl = pl.ds(off + g_ * 16, 16)
            ol = pl.ds(g_ * 16, 16)
            t16 = typ_v[sl]
            idxc_v[ol] = t16 * N + src_v[sl]
            dstc_v[ol] = dst_v[sl]
            if compute_w:
                keyc_v[ol] = dst_v[sl] * R + t16
            return c3
        lax.fori_loop(0, CHUNK // 16, lane, 0)
        pltpu.async_copy(h_hbm.at[idxc_v], rows_v, g)
        if compute_w:
            pltpu.async_copy(wd0_hbm.at[keyc_v], d0_v, s0)
            pltpu.async_copy(wd1_hbm.at[keyc_v], d1_v, s1)

    def process(off, idxc_v, dstc_v, keyc_v, rows_v, g, s0, s1, d0_v, d1_v):
        if compute_w:
            pltpu.make_async_copy(wd0_hbm.at[keyc_v], d0_v, s0).wait()
            pltpu.make_async_copy(wd1_hbm.at[keyc_v], d1_v, s1).wait()

            def lw(g_, c3):
                dsum = d0_v[pl.ds(g_ * 16, 16)] + d1_v[pl.ds(g_ * 16, 16)]
                w_v[pl.ds(off + g_ * 16, 16)] = 1.0 / jnp.maximum(dsum, 1.0)
                return c3
            lax.fori_loop(0, CHUNK // 16, lw, 0)
        pltpu.make_async_copy(h_hbm.at[idxc_v], rows_v, g).wait()

        def escale(g_, c3):
            wvec = w_v[pl.ds(off + g_ * 16, 16)]
            for i in range(16):
                wv = wvec[i]
                e = g_ * 16 + i
                for j in range(D // 16):
                    rows_v[e, pl.ds(j * 16, 16)] = (
                        rows_v[e, pl.ds(j * 16, 16)] * wv)
            return c3
        lax.fori_loop(0, CHUNK // 16, escale, 0)
        pltpu.sync_copy(rows_v, acc_sh.at[dstc_v], add=True)

    A = (idxc_a, dstc_a, keyc_a, rows_a, g_a, sd0_a, sd1_a, d0_a, d1_a)
    B = (idxc_b, dstc_b, keyc_b, rows_b, g_b, sd0_b, sd1_b, d0_b, d1_b)

    def fire_t(off, t):
        fire(off, t[0], t[1], t[2], t[3], t[4], t[5], t[6], t[7], t[8])

    def process_t(off, t):
        process(off, t[0], t[1], t[2], t[3], t[4], t[5], t[6], t[7], t[8])

    def sup(si, carry):
        sbase = base + si * SUP
        pltpu.sync_copy(src_hbm.at[pl.ds(sbase, SUP)], src_v)
        pltpu.sync_copy(dst_hbm.at[pl.ds(sbase, SUP)], dst_v)
        pltpu.sync_copy(typ_hbm.at[pl.ds(sbase, SUP)], typ_v)
        if not compute_w:
            pltpu.sync_copy(wd0_hbm.at[pl.ds(sbase, SUP)], w_v)

        fire_t(0, A)

        def pair(k, c2):
            fire_t((2 * k + 1) * CHUNK, B)
            process_t(2 * k * CHUNK, A)
            fire_t((2 * k + 2) * CHUNK, A)
            process_t((2 * k + 1) * CHUNK, B)
            return c2
        lax.fori_loop(0, (NCH_SUP - 1) // 2, pair, 0)
        process_t((NCH_SUP - 1) * CHUNK, A)

        if compute_w:
            pltpu.sync_copy(w_v, w_hbm.at[pl.ds(sbase, SUP)])
        return carry
    lax.fori_loop(0, NSUP, sup, 0)
    plsc.subcore_barrier()

    for j in range(NCP):
        row0 = s * ROWS_PER_TILE + j * CP_CHUNK
        pltpu.sync_copy(acc_sh.at[pl.ds(row0, CP_CHUNK)], cbuf_v)
        pltpu.sync_copy(cbuf_v, p_hbm.at[c, pl.ds(row0, CP_CHUNK)])


def _edge_aggregate(src, dst, typ, wd0, wd1, hflat, compute_w):
    return pl.kernel(
        functools.partial(_edges_body, compute_w),
        mesh=_mesh(),
        out_type=(
            jax.ShapeDtypeStruct((NC, N_PAD, D), jnp.float32),
            jax.ShapeDtypeStruct((E,), jnp.float32),
        ),
        scratch_types=[
            pltpu.VMEM((SUP,), jnp.int32),
            pltpu.VMEM((SUP,), jnp.int32),
            pltpu.VMEM((SUP,), jnp.int32),
            pltpu.VMEM((SUP,), jnp.float32),
            pltpu.VMEM((CHUNK,), jnp.int32),
            pltpu.VMEM((CHUNK,), jnp.int32),
            pltpu.VMEM((CHUNK,), jnp.int32),
            pltpu.VMEM((CHUNK,), jnp.int32),
            pltpu.VMEM((CHUNK,), jnp.int32),
            pltpu.VMEM((CHUNK,), jnp.int32),
            pltpu.VMEM((CHUNK,), jnp.float32),
            pltpu.VMEM((CHUNK,), jnp.float32),
            pltpu.VMEM((CHUNK,), jnp.float32),
            pltpu.VMEM((CHUNK,), jnp.float32),
            pltpu.VMEM((CHUNK, D), jnp.float32),
            pltpu.VMEM((CHUNK, D), jnp.float32),
            pltpu.VMEM((CP_CHUNK, D), jnp.float32),
            pltpu.VMEM_SHARED((N_PAD, D), jnp.float32),
            pltpu.SemaphoreType.DMA,
            pltpu.SemaphoreType.DMA,
            pltpu.SemaphoreType.DMA,
            pltpu.SemaphoreType.DMA,
            pltpu.SemaphoreType.DMA,
            pltpu.SemaphoreType.DMA,
        ],
    )(src, dst, typ, wd0, wd1, hflat)


def kernel(edge_index, edge_type, embedding,
           basis1, comp1, root1, bias1,
           basis2, comp2, root2, bias2):
    src = edge_index[0].astype(jnp.int32)
    dst = edge_index[1].astype(jnp.int32)
    typ = edge_type.astype(jnp.int32)

    comp_s = jnp.stack([comp1, comp2])
    basis_s = jnp.stack([basis1.reshape(NB, D * D),
                         basis2.reshape(NB, D * D)])
    w_mix = _mix_weights(comp_s, basis_s)
    w1 = w_mix[0].reshape(R, D, D).astype(jnp.bfloat16)
    w2 = w_mix[1].reshape(R, D, D).astype(jnp.bfloat16)

    deg = _deg_counts(dst, typ)

    h1 = _h_all_relations(embedding.astype(jnp.bfloat16), w1).reshape(R * N, D)
    p1, ew = _edge_aggregate(src, dst, typ, deg[0], deg[1], h1, compute_w=True)
    x2 = _combine(embedding, p1, root1, bias1.reshape(1, D), relu=True)

    h2 = _h_all_relations(x2.astype(jnp.bfloat16), w2).reshape(R * N, D)
    p2, _ = _edge_aggregate(src, dst, typ, ew, ew, h2, compute_w=False)
    out = _combine(x2, p2, root2, bias2.reshape(1, D), relu=False)
    return out


# speedup vs baseline: 3.2985x; 1.1558x over previous
import functools

import jax
import jax.numpy as jnp
from jax import lax
from jax.experimental import pallas as pl
from jax.experimental.pallas import tpu as pltpu
from jax.experimental.pallas import tpu_sc as plsc

N = 10000
R = 50
D = 128
NB = 30
E = 320000

NC = 2
NS = 16
NW = NC * NS

E_PER_TILE = E // NW
CHUNK = 80
NCHUNK = E_PER_TILE // CHUNK

NSEG = N * R
NSEG_PAD = 512000
SEG_PER_TILE = NSEG_PAD // NS

N_PAD = 10240
ROWS_PER_TILE = N_PAD // NS
CP_CHUNK = 64
NCP = ROWS_PER_TILE // CP_CHUNK

SUP = 2000
NSUP = E_PER_TILE // SUP
NCH_SUP = SUP // CHUNK

NT_BLK = 1000
NT_GRID = N // NT_BLK

@functools.lru_cache(maxsize=None)
def _mesh():
    return plsc.VectorSubcoreMesh(core_axis_name="c", subcore_axis_name="s")


def _h1_body(x_ref, comp_ref, basis_ref, h_ref, wscr, xbscr):
    i = pl.program_id(0)
    r = pl.program_id(1)

    @pl.when(jnp.logical_and(i == 0, r == 0))
    def _():
        wmix = jnp.dot(comp_ref[...], basis_ref[...],
                       preferred_element_type=jnp.float32)
        wscr[...] = wmix.reshape(R * D, D).astype(jnp.bfloat16)

    @pl.when(r == 0)
    def _():
        xbscr[...] = x_ref[...].astype(jnp.bfloat16)

    h_ref[0] = jnp.dot(xbscr[...], wscr[pl.ds(r * D, D), :],
                       preferred_element_type=jnp.float32)


def _h_layer1(x, comp, basisflat):
    return pl.pallas_call(
        _h1_body,
        grid=(NT_GRID, R),
        in_specs=[
            pl.BlockSpec((NT_BLK, D), lambda i, r: (i, 0)),
            pl.BlockSpec((R, NB), lambda i, r: (0, 0)),
            pl.BlockSpec((NB, D * D), lambda i, r: (0, 0)),
        ],
        out_specs=pl.BlockSpec((1, NT_BLK, D), lambda i, r: (r, i, 0)),
        out_shape=jax.ShapeDtypeStruct((R, N, D), jnp.float32),
        scratch_shapes=[
            pltpu.VMEM((R * D, D), jnp.bfloat16),
            pltpu.VMEM((NT_BLK, D), jnp.bfloat16),
        ],
    )(x, comp, basisflat)


def _h2_body(x_ref, p_ref, root_ref, bias_ref, comp_ref, basis_ref,
             h_ref, x2_ref, wscr, xbscr):
    i = pl.program_id(0)
    r = pl.program_id(1)

    @pl.when(jnp.logical_and(i == 0, r == 0))
    def _():
        wmix = jnp.dot(comp_ref[...], basis_ref[...],
                       preferred_element_type=jnp.float32)
        wscr[...] = wmix.reshape(R * D, D).astype(jnp.bfloat16)

    @pl.when(r == 0)
    def _():
        acc = p_ref[0] + p_ref[1] + bias_ref[0]
        acc = acc + jnp.dot(x_ref[...], root_ref[...],
                            preferred_element_type=jnp.float32)
        acc = jnp.maximum(acc, 0.0)
        x2_ref[...] = acc
        xbscr[...] = acc.astype(jnp.bfloat16)

    h_ref[0] = jnp.dot(xbscr[...], wscr[pl.ds(r * D, D), :],
                       preferred_element_type=jnp.float32)


def _h_layer2(x, p, root, bias2d, comp, basisflat):
    return pl.pallas_call(
        _h2_body,
        grid=(NT_GRID, R),
        in_specs=[
            pl.BlockSpec((NT_BLK, D), lambda i, r: (i, 0)),
            pl.BlockSpec((2, NT_BLK, D), lambda i, r: (0, i, 0)),
            pl.BlockSpec((D, D), lambda i, r: (0, 0)),
            pl.BlockSpec((1, D), lambda i, r: (0, 0)),
            pl.BlockSpec((R, NB), lambda i, r: (0, 0)),
            pl.BlockSpec((NB, D * D), lambda i, r: (0, 0)),
        ],
        out_specs=[
            pl.BlockSpec((1, NT_BLK, D), lambda i, r: (r, i, 0)),
            pl.BlockSpec((NT_BLK, D), lambda i, r: (i, 0)),
        ],
        out_shape=[
            jax.ShapeDtypeStruct((R, N, D), jnp.float32),
            jax.ShapeDtypeStruct((N, D), jnp.float32),
        ],
        scratch_shapes=[
            pltpu.VMEM((R * D, D), jnp.bfloat16),
            pltpu.VMEM((NT_BLK, D), jnp.bfloat16),
        ],
    )(x, p, root, bias2d, comp, basisflat)


def _combine_body(x_ref, p_ref, root_ref, bias_ref, o_ref):
    acc = p_ref[0] + p_ref[1] + bias_ref[0]
    acc = acc + jnp.dot(x_ref[...], root_ref[...],
                        preferred_element_type=jnp.float32)
    o_ref[...] = acc


def _combine(x, p, root, bias2d):
    return pl.pallas_call(
        _combine_body,
        grid=(NT_GRID,),
        in_specs=[
            pl.BlockSpec((NT_BLK, D), lambda i: (i, 0)),
            pl.BlockSpec((2, NT_BLK, D), lambda i: (0, i, 0)),
            pl.BlockSpec((D, D), lambda i: (0, 0)),
            pl.BlockSpec((1, D), lambda i: (0, 0)),
        ],
        out_specs=pl.BlockSpec((NT_BLK, D), lambda i: (i, 0)),
        out_shape=jax.ShapeDtypeStruct((N, D), jnp.float32),
    )(x, p, root, bias2d)


def _zero_vmem(ref, nelem):
    def body(i, carry):
        ref[pl.ds(i * 16, 16)] = jnp.zeros((16,), jnp.float32)
        return carry
    lax.fori_loop(0, nelem // 16, body, 0)


def _deg_body(dst_hbm, typ_hbm, deg_hbm, deg1_hbm,
              dst_v, typ_v, key_v, ones_v, buf_v, acc_sh, sem):
    c = lax.axis_index("c")
    s = lax.axis_index("s")
    _zero_vmem(buf_v, SEG_PER_TILE)
    pltpu.sync_copy(buf_v, acc_sh.at[pl.ds(s * SEG_PER_TILE, SEG_PER_TILE)])

    def ob(i, carry):
        ones_v[pl.ds(i * 16, 16)] = jnp.ones((16,), jnp.float32)
        return carry
    lax.fori_loop(0, CHUNK // 16, ob, 0)
    plsc.subcore_barrier()

    base = (c * NS + s) * E_PER_TILE
    pltpu.sync_copy(dst_hbm.at[pl.ds(base, E_PER_TILE)], dst_v)
    pltpu.sync_copy(typ_hbm.at[pl.ds(base, E_PER_TILE)], typ_v)

    def chunk(ci, carry):
        off = ci * CHUNK

        def lane(g, carry2):
            key_v[pl.ds(g * 16, 16)] = (dst_v[pl.ds(off + g * 16, 16)] * R
                                        + typ_v[pl.ds(off + g * 16, 16)])
            return carry2
        lax.fori_loop(0, CHUNK // 16, lane, 0)
        pltpu.sync_copy(ones_v, acc_sh.at[key_v], add=True)
        return carry
    lax.fori_loop(0, NCHUNK, chunk, 0)
    plsc.subcore_barrier()

    pltpu.sync_copy(acc_sh.at[pl.ds(s * SEG_PER_TILE, SEG_PER_TILE)], buf_v)

    @pl.when(c == 0)
    def _():
        pltpu.sync_copy(buf_v, deg_hbm.at[pl.ds(s * SEG_PER_TILE, SEG_PER_TILE)])

    @pl.when(c == 1)
    def _():
        pltpu.sync_copy(buf_v, deg1_hbm.at[pl.ds(s * SEG_PER_TILE, SEG_PER_TILE)])


def _deg_counts(dst, typ):
    return pl.kernel(
        _deg_body,
        mesh=_mesh(),
        out_type=(jax.ShapeDtypeStruct((NSEG_PAD,), jnp.float32),
                  jax.ShapeDtypeStruct((NSEG_PAD,), jnp.float32)),
        scratch_types=[
            pltpu.VMEM((E_PER_TILE,), jnp.int32),
            pltpu.VMEM((E_PER_TILE,), jnp.int32),
            pltpu.VMEM((CHUNK,), jnp.int32),
            pltpu.VMEM((CHUNK,), jnp.float32),
            pltpu.VMEM((SEG_PER_TILE,), jnp.float32),
            pltpu.VMEM_SHARED((NSEG_PAD,), jnp.float32),
            pltpu.SemaphoreType.DMA,
        ],
    )(dst, typ)


def _edges_body(compute_w, src_hbm, dst_hbm, typ_hbm, wd0_hbm, wd1_hbm, h_hbm,
                p_hbm, w_hbm,
                src_v, dst_v, typ_v, w_v,
                idxc_a, idxc_b, dstc_a, dstc_b, keyc_a, keyc_b,
                d0_a, d1_a, d0_b, d1_b, rows_a, rows_b, cbuf_v, acc_sh,
                g_a, g_b, sd0_a, sd1_a, sd0_b, sd1_b):
    c = lax.axis_index("c")
    s = lax.axis_index("s")
    base = (c * NS + s) * E_PER_TILE

    def zrow(i, carry):
        for j in range(D // 16):
            cbuf_v[i, pl.ds(j * 16, 16)] = jnp.zeros((16,), jnp.float32)
        return carry
    lax.fori_loop(0, CP_CHUNK, zrow, 0)
    for j in range(NCP):
        pltpu.sync_copy(
            cbuf_v,
            acc_sh.at[pl.ds(s * ROWS_PER_TILE + j * CP_CHUNK, CP_CHUNK)])
    plsc.subcore_barrier()

    def fire(off, idxc_v, dstc_v, keyc_v, rows_v, g, s0, s1, d0_v, d1_v):
        def lane(g_, c3):
            sl = pl.ds(off + g_ * 16, 16)
            ol = pl.ds(g_ * 16, 16)
            t16 = typ_v[sl]
            idxc_v[ol] = t16 * N + src_v[sl]
            dstc_v[ol] = dst_v[sl]
            if compute_w:
                keyc_v[ol] = dst_v[sl] * R + t16
            return c3
        lax.fori_loop(0, CHUNK // 16, lane, 0)
        pltpu.async_copy(h_hbm.at[idxc_v], rows_v, g)
        if compute_w:
            pltpu.async_copy(wd0_hbm.at[keyc_v], d0_v, s0)
            pltpu.async_copy(wd1_hbm.at[keyc_v], d1_v, s1)

    def process(off, idxc_v, dstc_v, keyc_v, rows_v, g, s0, s1, d0_v, d1_v):
        if compute_w:
            pltpu.make_async_copy(wd0_hbm.at[keyc_v], d0_v, s0).wait()
            pltpu.make_async_copy(wd1_hbm.at[keyc_v], d1_v, s1).wait()

            def lw(g_, c3):
                dsum = d0_v[pl.ds(g_ * 16, 16)] + d1_v[pl.ds(g_ * 16, 16)]
                w_v[pl.ds(off + g_ * 16, 16)] = 1.0 / jnp.maximum(dsum, 1.0)
                return c3
            lax.fori_loop(0, CHUNK // 16, lw, 0)
        pltpu.make_async_copy(h_hbm.at[idxc_v], rows_v, g).wait()

        def escale(g_, c3):
            wvec = w_v[pl.ds(off + g_ * 16, 16)]
            for i in range(16):
                wv = wvec[i]
                e = g_ * 16 + i
                for j in range(D // 16):
                    rows_v[e, pl.ds(j * 16, 16)] = (
                        rows_v[e, pl.ds(j * 16, 16)] * wv)
            return c3
        lax.fori_loop(0, CHUNK // 16, escale, 0)
        pltpu.sync_copy(rows_v, acc_sh.at[dstc_v], add=True)

    A = (idxc_a, dstc_a, keyc_a, rows_a, g_a, sd0_a, sd1_a, d0_a, d1_a)
    B = (idxc_b, dstc_b, keyc_b, rows_b, g_b, sd0_b, sd1_b, d0_b, d1_b)

    def fire_t(off, t):
        fire(off, t[0], t[1], t[2], t[3], t[4], t[5], t[6], t[7], t[8])

    def process_t(off, t):
        process(off, t[0], t[1], t[2], t[3], t[4], t[5], t[6], t[7], t[8])

    def sup(si, carry):
        sbase = base + si * SUP
        pltpu.sync_copy(src_hbm.at[pl.ds(sbase, SUP)], src_v)
        pltpu.sync_copy(dst_hbm.at[pl.ds(sbase, SUP)], dst_v)
        pltpu.sync_copy(typ_hbm.at[pl.ds(sbase, SUP)], typ_v)
        if not compute_w:
            pltpu.sync_copy(wd0_hbm.at[pl.ds(sbase, SUP)], w_v)

        fire_t(0, A)

        def pair(k, c2):
            fire_t((2 * k + 1) * CHUNK, B)
            process_t(2 * k * CHUNK, A)
            fire_t((2 * k + 2) * CHUNK, A)
            process_t((2 * k + 1) * CHUNK, B)
            return c2
        lax.fori_loop(0, (NCH_SUP - 1) // 2, pair, 0)
        process_t((NCH_SUP - 1) * CHUNK, A)

        if compute_w:
            pltpu.sync_copy(w_v, w_hbm.at[pl.ds(sbase, SUP)])
        return carry
    lax.fori_loop(0, NSUP, sup, 0)
    plsc.subcore_barrier()

    for j in range(NCP):
        row0 = s * ROWS_PER_TILE + j * CP_CHUNK
        pltpu.sync_copy(acc_sh.at[pl.ds(row0, CP_CHUNK)], cbuf_v)
        pltpu.sync_copy(cbuf_v, p_hbm.at[c, pl.ds(row0, CP_CHUNK)])


def _edge_aggregate(src, dst, typ, wd0, wd1, hflat, compute_w):
    return pl.kernel(
        functools.partial(_edges_body, compute_w),
        mesh=_mesh(),
        out_type=(
            jax.ShapeDtypeStruct((NC, N_PAD, D), jnp.float32),
            jax.ShapeDtypeStruct((E,), jnp.float32),
        ),
        scratch_types=[
            pltpu.VMEM((SUP,), jnp.int32),
            pltpu.VMEM((SUP,), jnp.int32),
            pltpu.VMEM((SUP,), jnp.int32),
            pltpu.VMEM((SUP,), jnp.float32),
            pltpu.VMEM((CHUNK,), jnp.int32),
            pltpu.VMEM((CHUNK,), jnp.int32),
            pltpu.VMEM((CHUNK,), jnp.int32),
            pltpu.VMEM((CHUNK,), jnp.int32),
            pltpu.VMEM((CHUNK,), jnp.int32),
            pltpu.VMEM((CHUNK,), jnp.int32),
            pltpu.VMEM((CHUNK,), jnp.float32),
            pltpu.VMEM((CHUNK,), jnp.float32),
            pltpu.VMEM((CHUNK,), jnp.float32),
            pltpu.VMEM((CHUNK,), jnp.float32),
            pltpu.VMEM((CHUNK, D), jnp.float32),
            pltpu.VMEM((CHUNK, D), jnp.float32),
            pltpu.VMEM((CP_CHUNK, D), jnp.float32),
            pltpu.VMEM_SHARED((N_PAD, D), jnp.float32),
            pltpu.SemaphoreType.DMA,
            pltpu.SemaphoreType.DMA,
            pltpu.SemaphoreType.DMA,
            pltpu.SemaphoreType.DMA,
            pltpu.SemaphoreType.DMA,
            pltpu.SemaphoreType.DMA,
        ],
    )(src, dst, typ, wd0, wd1, hflat)


def kernel(edge_index, edge_type, embedding,
           basis1, comp1, root1, bias1,
           basis2, comp2, root2, bias2):
    src = edge_index[0].astype(jnp.int32)
    dst = edge_index[1].astype(jnp.int32)
    typ = edge_type.astype(jnp.int32)
    basis1f = basis1.reshape(NB, D * D)
    basis2f = basis2.reshape(NB, D * D)

    d0, d1 = _deg_counts(dst, typ)

    h1 = _h_layer1(embedding, comp1, basis1f).reshape(R * N, D)
    p1, ew = _edge_aggregate(src, dst, typ, d0, d1, h1, compute_w=True)

    h2, x2 = _h_layer2(embedding, p1, root1, bias1.reshape(1, D), comp2, basis2f)
    h2 = h2.reshape(R * N, D)
    p2, _ = _edge_aggregate(src, dst, typ, ew, ew, h2, compute_w=False)
    out = _combine(x2, p2, root2, bias2.reshape(1, D))
    return out
